# Initial kernel scaffold; baseline (speedup 1.0000x reference)
import jax
import jax.numpy as jnp
from jax.experimental import pallas as pl


def kernel(x, edge_index, W1, b1, W2, b2, Wl, bl):
    raise NotImplementedError("write your pallas kernel here")


# speedup vs baseline: 8.5957x; 8.5957x over previous
import functools

import jax
import jax.numpy as jnp
from jax import lax
from jax.experimental import pallas as pl
from jax.experimental.pallas import tpu as pltpu
from jax.experimental.pallas import tpu_sc as plsc

N_NODES = 100000
N_EDGES = 3200000
D = 16

NC = 2
NS = 16
NW = NC * NS

N_PAD = 100352
E_PAD = N_PAD * NW
SUB = 64
ROWS = E_PAD // SUB
RPW = ROWS // NW
ZROW = N_PAD // NS

BM = 2048
GRID = N_PAD // BM

_MESH = plsc.VectorSubcoreMesh(
    core_axis_name="c", subcore_axis_name="s", num_cores=NC, num_subcores=NS
)


@functools.partial(
    pl.kernel,
    mesh=_MESH,
    out_type=jax.ShapeDtypeStruct((NC, N_PAD, D), jnp.float32),
    compiler_params=pltpu.CompilerParams(needs_layout_passes=False),
    scratch_types=[
        pltpu.VMEM((1, SUB), jnp.int32),
        pltpu.VMEM((SUB, D), jnp.float32),
        pltpu.VMEM((SUB, D), jnp.float32),
        pltpu.VMEM((1, SUB), jnp.int32),
        pltpu.VMEM_SHARED((N_PAD, D), jnp.float32),
        pltpu.SemaphoreType.DMA,
    ],
)
def _sc_degree(dst2_hbm, out_hbm, dst_v, buf_v, zbuf_v, iid_v, deg_sh, sem):
    cid = lax.axis_index("c")
    sid = lax.axis_index("s")
    wid = sid * NC + cid
    iota16 = lax.iota(jnp.int32, 16)

    for i in range(SUB):
        buf_v[i] = jnp.ones((16,), jnp.float32)
        zbuf_v[i] = jnp.zeros((16,), jnp.float32)
    base = sid * ZROW

    @pl.loop(0, ZROW // SUB)
    def _init(t):
        r = base + t * SUB
        for k in range(SUB // 16):
            iid_v[0, pl.ds(k * 16, 16)] = iota16 + (r + k * 16)
        pltpu.sync_copy(zbuf_v, deg_sh.at[iid_v.at[0]])

    plsc.subcore_barrier()

    @pl.loop(0, RPW)
    def _body(g):
        r0 = wid * RPW + g
        pltpu.sync_copy(dst2_hbm.at[pl.ds(r0, 1)], dst_v)
        pltpu.sync_copy(buf_v, deg_sh.at[dst_v.at[0]], add=True)

    plsc.subcore_barrier()

    @pl.loop(0, ZROW // SUB)
    def _read(t):
        r = base + t * SUB
        for k in range(SUB // 16):
            iid_v[0, pl.ds(k * 16, 16)] = iota16 + (r + k * 16)
        pltpu.async_copy(deg_sh.at[iid_v.at[0]], zbuf_v, sem).wait()
        pltpu.sync_copy(zbuf_v, out_hbm.at[cid, pl.ds(r, SUB)])


@functools.partial(
    pl.kernel,
    mesh=_MESH,
    out_type=jax.ShapeDtypeStruct((E_PAD, D), jnp.float32),
    compiler_params=pltpu.CompilerParams(needs_layout_passes=False),
    scratch_types=[
        pltpu.VMEM((1, SUB), jnp.int32),
        pltpu.VMEM((SUB, D), jnp.float32),
        pltpu.VMEM((1, SUB), jnp.int32),
        pltpu.VMEM_SHARED((N_PAD, D), jnp.float32),
        pltpu.SemaphoreType.DMA,
    ],
)
def _sc_gather(hs_hbm, src2_hbm, msg_hbm, idx_v, buf_v, iid_v, hs_sh, sem):
    cid = lax.axis_index("c")
    sid = lax.axis_index("s")
    wid = sid * NC + cid
    iota16 = lax.iota(jnp.int32, 16)
    base = sid * ZROW

    @pl.loop(0, ZROW // SUB)
    def _load(t):
        r = base + t * SUB
        pltpu.sync_copy(hs_hbm.at[pl.ds(r, SUB)], buf_v)
        for k in range(SUB // 16):
            iid_v[0, pl.ds(k * 16, 16)] = iota16 + (r + k * 16)
        pltpu.sync_copy(buf_v, hs_sh.at[iid_v.at[0]])

    plsc.subcore_barrier()

    @pl.loop(0, RPW)
    def _body(g):
        r0 = wid * RPW + g
        pltpu.sync_copy(src2_hbm.at[pl.ds(r0, 1)], idx_v)
        pltpu.async_copy(hs_sh.at[idx_v.at[0]], buf_v, sem).wait()
        pltpu.sync_copy(buf_v, msg_hbm.at[pl.ds(r0 * SUB, SUB)])


@functools.partial(
    pl.kernel,
    mesh=_MESH,
    out_type=jax.ShapeDtypeStruct((NC, N_PAD, D), jnp.float32),
    compiler_params=pltpu.CompilerParams(needs_layout_passes=False),
    scratch_types=[
        pltpu.VMEM((1, SUB), jnp.int32),
        pltpu.VMEM((SUB, D), jnp.float32),
        pltpu.VMEM((SUB, D), jnp.float32),
        pltpu.VMEM((1, SUB), jnp.int32),
        pltpu.VMEM_SHARED((N_PAD, D), jnp.float32),
        pltpu.SemaphoreType.DMA,
    ],
)
def _sc_accum(msg_hbm, dst2_hbm, out_hbm, idx_v, buf_v, zbuf_v, iid_v, acc_sh, sem):
    cid = lax.axis_index("c")
    sid = lax.axis_index("s")
    wid = sid * NC + cid
    iota16 = lax.iota(jnp.int32, 16)

    for i in range(SUB):
        zbuf_v[i] = jnp.zeros((16,), jnp.float32)
    base = sid * ZROW

    @pl.loop(0, ZROW // SUB)
    def _init(t):
        r = base + t * SUB
        for k in range(SUB // 16):
            iid_v[0, pl.ds(k * 16, 16)] = iota16 + (r + k * 16)
        pltpu.sync_copy(zbuf_v, acc_sh.at[iid_v.at[0]])

    plsc.subcore_barrier()

    @pl.loop(0, RPW)
    def _body(g):
        r0 = wid * RPW + g
        pltpu.sync_copy(dst2_hbm.at[pl.ds(r0, 1)], idx_v)
        pltpu.sync_copy(msg_hbm.at[pl.ds(r0 * SUB, SUB)], buf_v)
        pltpu.sync_copy(buf_v, acc_sh.at[idx_v.at[0]], add=True)

    plsc.subcore_barrier()

    @pl.loop(0, ZROW // SUB)
    def _read(t):
        r = base + t * SUB
        for k in range(SUB // 16):
            iid_v[0, pl.ds(k * 16, 16)] = iota16 + (r + k * 16)
        pltpu.async_copy(acc_sh.at[iid_v.at[0]], zbuf_v, sem).wait()
        pltpu.sync_copy(zbuf_v, out_hbm.at[cid, pl.ds(r, SUB)])


def _tc_a_body(x_ref, degt_ref, w1_ref, hs_ref, dis_ref):
    deg = degt_ref[0, :, 0:1] + degt_ref[1, :, 0:1] + 1.0
    dis = lax.rsqrt(deg)
    h = jnp.dot(x_ref[...], w1_ref[...], preferred_element_type=jnp.float32)
    hs_ref[...] = h * dis
    dis_ref[...] = dis


def _tc_b_body(acc_ref, hs_ref, dis_ref, b1_ref, w2_ref, hs2_ref):
    dis = dis_ref[...]
    o1 = dis * (acc_ref[0] + acc_ref[1] + hs_ref[...]) + b1_ref[...]
    o1 = jnp.maximum(o1, 0.0)
    h2 = jnp.dot(o1, w2_ref[...], preferred_element_type=jnp.float32)
    hs2_ref[...] = h2 * dis


def _tc_c_body(acc_ref, hs2_ref, dis_ref, b2_ref, wl_ref, bl_ref, y_ref):
    dis = dis_ref[...]
    o2 = dis * (acc_ref[0] + acc_ref[1] + hs2_ref[...]) + b2_ref[...]
    y_ref[...] = jnp.dot(o2, wl_ref[...], preferred_element_type=jnp.float32) + bl_ref[...]


def _row_spec(width):
    return pl.BlockSpec((BM, width), lambda i: (i, 0))


def _full_spec(shape):
    return pl.BlockSpec(shape, lambda i: tuple(0 for _ in shape))


_acc_spec = pl.BlockSpec((NC, BM, D), lambda i: (0, i, 0))

_tc_a = pl.pallas_call(
    _tc_a_body,
    grid=(GRID,),
    in_specs=[_row_spec(D), _acc_spec, _full_spec((D, D))],
    out_specs=[_row_spec(D), _row_spec(1)],
    out_shape=[
        jax.ShapeDtypeStruct((N_PAD, D), jnp.float32),
        jax.ShapeDtypeStruct((N_PAD, 1), jnp.float32),
    ],
)

_tc_b = pl.pallas_call(
    _tc_b_body,
    grid=(GRID,),
    in_specs=[_acc_spec, _row_spec(D), _row_spec(1), _full_spec((1, D)), _full_spec((D, D))],
    out_specs=_row_spec(D),
    out_shape=jax.ShapeDtypeStruct((N_PAD, D), jnp.float32),
)

_tc_c = pl.pallas_call(
    _tc_c_body,
    grid=(GRID,),
    in_specs=[
        _acc_spec,
        _row_spec(D),
        _row_spec(1),
        _full_spec((1, D)),
        _full_spec((D, 1)),
        _full_spec((1, 1)),
    ],
    out_specs=_row_spec(1),
    out_shape=jax.ShapeDtypeStruct((N_PAD, 1), jnp.float32),
)


def kernel(x, edge_index, W1, b1, W2, b2, Wl, bl):
    ei = edge_index.astype(jnp.int32)
    src = ei[0]
    dst = ei[1]
    npad = E_PAD - N_EDGES
    fill = N_NODES + (jnp.arange(npad, dtype=jnp.int32) % 256)
    src2 = jnp.concatenate([src, fill]).reshape(ROWS, SUB)
    dst2 = jnp.concatenate([dst, fill]).reshape(ROWS, SUB)
    x_p = jnp.concatenate([x, jnp.zeros((N_PAD - N_NODES, D), jnp.float32)])

    degp = _sc_degree(dst2)
    hs1, dis = _tc_a(x_p, degp, W1)
    acc1 = _sc_accum(_sc_gather(hs1, src2), dst2)
    hs2 = _tc_b(acc1, hs1, dis, b1.reshape(1, D), W2)
    acc2 = _sc_accum(_sc_gather(hs2, src2), dst2)
    y = _tc_c(acc2, hs2, dis, b2.reshape(1, D), Wl, bl.reshape(1, 1))
    return y[:N_NODES]
